# baseline (speedup 1.0000x reference)
import functools

import jax
import jax.numpy as jnp
from jax.experimental import pallas as pl
from jax.experimental.pallas import tpu as pltpu


def _kernel(x_ref, o_ref, carry_ref, *, W):
    j = pl.program_id(1)

    @pl.when(j == 0)
    def _():
        carry_ref[...] = jnp.zeros_like(carry_ref)

    x = x_ref[...]
    rows = jax.lax.broadcasted_iota(jnp.int32, (W, W), 0)
    cols = jax.lax.broadcasted_iota(jnp.int32, (W, W), 1)
    tri = (rows >= cols).astype(jnp.float32)
    rc = jax.lax.dot_general(
        x, tri, (((1,), (0,)), ((), ())),
        preferred_element_type=jnp.float32,
        precision=jax.lax.Precision.DEFAULT,
    )
    o_ref[...] = rc + carry_ref[...]
    carry_ref[...] = carry_ref[...] + rc[:, 0:1]


def kernel(x):
    M, N = x.shape
    BR, W = 256, 256
    ncb = N // W
    grid = (M // BR, ncb)
    return pl.pallas_call(
        functools.partial(_kernel, W=W),
        grid=grid,
        in_specs=[pl.BlockSpec((BR, W), lambda i, j: (i, ncb - 1 - j))],
        out_specs=pl.BlockSpec((BR, W), lambda i, j: (i, ncb - 1 - j)),
        out_shape=jax.ShapeDtypeStruct((M, N), x.dtype),
        scratch_shapes=[pltpu.VMEM((BR, 1), jnp.float32)],
    )(x)


# speedup vs baseline: 1.6806x; 1.6806x over previous
import functools

import jax
import jax.numpy as jnp
from jax.experimental import pallas as pl
from jax.experimental.pallas import tpu as pltpu


def _kernel(x_ref, o_ref, carry_ref, *, W):
    j = pl.program_id(1)

    @pl.when(j == 0)
    def _():
        carry_ref[...] = jnp.zeros_like(carry_ref)

    x = x_ref[...]
    rows = jax.lax.broadcasted_iota(jnp.int32, (W, W), 0)
    cols = jax.lax.broadcasted_iota(jnp.int32, (W, W), 1)
    tri = (rows >= cols).astype(jnp.float32)
    rc = jax.lax.dot_general(
        x, tri, (((1,), (0,)), ((), ())),
        preferred_element_type=jnp.float32,
        precision=jax.lax.Precision.DEFAULT,
    )
    o_ref[...] = rc + carry_ref[...]
    carry_ref[...] = carry_ref[...] + rc[:, 0:1]


def kernel(x):
    M, N = x.shape
    BR, W = 512, 256
    ncb = N // W
    grid = (M // BR, ncb)
    return pl.pallas_call(
        functools.partial(_kernel, W=W),
        grid=grid,
        in_specs=[pl.BlockSpec((BR, W), lambda i, j: (i, ncb - 1 - j))],
        out_specs=pl.BlockSpec((BR, W), lambda i, j: (i, ncb - 1 - j)),
        out_shape=jax.ShapeDtypeStruct((M, N), x.dtype),
        scratch_shapes=[pltpu.VMEM((BR, 1), jnp.float32)],
    )(x)


# speedup vs baseline: 2.4477x; 1.4564x over previous
import functools

import jax
import jax.numpy as jnp
from jax.experimental import pallas as pl
from jax.experimental.pallas import tpu as pltpu


def _kernel(x_ref, o_ref, carry_ref, *, W):
    j = pl.program_id(1)

    @pl.when(j == 0)
    def _():
        carry_ref[...] = jnp.zeros_like(carry_ref)

    x = x_ref[...]
    rows = jax.lax.broadcasted_iota(jnp.int32, (W, W), 0)
    cols = jax.lax.broadcasted_iota(jnp.int32, (W, W), 1)
    tri = (rows >= cols).astype(jnp.float32)
    rc = jax.lax.dot_general(
        x, tri, (((1,), (0,)), ((), ())),
        preferred_element_type=jnp.float32,
        precision=jax.lax.Precision.DEFAULT,
    )
    o_ref[...] = rc + carry_ref[...]
    carry_ref[...] = carry_ref[...] + rc[:, 0:1]


def kernel(x):
    M, N = x.shape
    BR, W = 1024, 256
    ncb = N // W
    grid = (M // BR, ncb)
    return pl.pallas_call(
        functools.partial(_kernel, W=W),
        grid=grid,
        in_specs=[pl.BlockSpec((BR, W), lambda i, j: (i, ncb - 1 - j))],
        out_specs=pl.BlockSpec((BR, W), lambda i, j: (i, ncb - 1 - j)),
        out_shape=jax.ShapeDtypeStruct((M, N), x.dtype),
        scratch_shapes=[pltpu.VMEM((BR, 1), jnp.float32)],
    )(x)


# speedup vs baseline: 3.5147x; 1.4359x over previous
import functools

import jax
import jax.numpy as jnp
from jax.experimental import pallas as pl
from jax.experimental.pallas import tpu as pltpu


def _kernel(x_ref, o_ref, carry_ref, *, W):
    j = pl.program_id(1)

    @pl.when(j == 0)
    def _():
        carry_ref[...] = jnp.zeros_like(carry_ref)

    x = x_ref[...]
    rows = jax.lax.broadcasted_iota(jnp.int32, (W, W), 0)
    cols = jax.lax.broadcasted_iota(jnp.int32, (W, W), 1)
    tri = (rows >= cols).astype(jnp.float32)
    rc = jax.lax.dot_general(
        x, tri, (((1,), (0,)), ((), ())),
        preferred_element_type=jnp.float32,
        precision=jax.lax.Precision.DEFAULT,
    )
    o_ref[...] = rc + carry_ref[...]
    carry_ref[...] = carry_ref[...] + rc[:, 0:1]


def kernel(x):
    M, N = x.shape
    BR, W = 2048, 256
    ncb = N // W
    grid = (M // BR, ncb)
    return pl.pallas_call(
        functools.partial(_kernel, W=W),
        grid=grid,
        in_specs=[pl.BlockSpec((BR, W), lambda i, j: (i, ncb - 1 - j))],
        out_specs=pl.BlockSpec((BR, W), lambda i, j: (i, ncb - 1 - j)),
        out_shape=jax.ShapeDtypeStruct((M, N), x.dtype),
        scratch_shapes=[pltpu.VMEM((BR, 1), jnp.float32)],
    )(x)


# speedup vs baseline: 4.0224x; 1.1444x over previous
import functools

import jax
import jax.numpy as jnp
from jax.experimental import pallas as pl
from jax.experimental.pallas import tpu as pltpu


def _kernel(x_ref, o_ref, carry_ref, *, W):
    j = pl.program_id(1)

    @pl.when(j == 0)
    def _():
        carry_ref[...] = jnp.zeros_like(carry_ref)

    x = x_ref[...]
    rows = jax.lax.broadcasted_iota(jnp.int32, (W, W), 0)
    cols = jax.lax.broadcasted_iota(jnp.int32, (W, W), 1)
    tri = (rows >= cols).astype(jnp.float32)
    rc = jax.lax.dot_general(
        x, tri, (((1,), (0,)), ((), ())),
        preferred_element_type=jnp.float32,
        precision=jax.lax.Precision.DEFAULT,
    )
    o_ref[...] = rc + carry_ref[...]
    carry_ref[...] = carry_ref[...] + rc[:, 0:1]


def kernel(x):
    M, N = x.shape
    BR, W = 4096, 256
    ncb = N // W
    grid = (M // BR, ncb)
    return pl.pallas_call(
        functools.partial(_kernel, W=W),
        grid=grid,
        in_specs=[pl.BlockSpec((BR, W), lambda i, j: (i, ncb - 1 - j))],
        out_specs=pl.BlockSpec((BR, W), lambda i, j: (i, ncb - 1 - j)),
        out_shape=jax.ShapeDtypeStruct((M, N), x.dtype),
        scratch_shapes=[pltpu.VMEM((BR, 1), jnp.float32)],
    )(x)


# speedup vs baseline: 4.0561x; 1.0084x over previous
import functools

import jax
import jax.numpy as jnp
from jax.experimental import pallas as pl
from jax.experimental.pallas import tpu as pltpu


def _kernel(x_ref, o_ref, carry_ref, *, W):
    j = pl.program_id(1)

    @pl.when(j == 0)
    def _():
        carry_ref[...] = jnp.zeros_like(carry_ref)

    x = x_ref[...]
    rows = jax.lax.broadcasted_iota(jnp.int32, (W, W), 0)
    cols = jax.lax.broadcasted_iota(jnp.int32, (W, W), 1)
    tri = (rows >= cols).astype(jnp.float32)
    rc = jax.lax.dot_general(
        x, tri, (((1,), (0,)), ((), ())),
        preferred_element_type=jnp.float32,
        precision=jax.lax.Precision.DEFAULT,
    )
    o_ref[...] = rc + carry_ref[...]
    carry_ref[...] = carry_ref[...] + rc[:, 0:1]


def kernel(x):
    M, N = x.shape
    BR, W = 4096, 512
    ncb = N // W
    grid = (M // BR, ncb)
    return pl.pallas_call(
        functools.partial(_kernel, W=W),
        grid=grid,
        in_specs=[pl.BlockSpec((BR, W), lambda i, j: (i, ncb - 1 - j))],
        out_specs=pl.BlockSpec((BR, W), lambda i, j: (i, ncb - 1 - j)),
        out_shape=jax.ShapeDtypeStruct((M, N), x.dtype),
        scratch_shapes=[pltpu.VMEM((BR, 1), jnp.float32)],
    )(x)
